# baseline (speedup 1.0000x reference)
import functools

import jax
import jax.numpy as jnp
from jax import lax
from jax.experimental import pallas as pl
from jax.experimental.pallas import tpu as pltpu
from jax.experimental.pallas import tpu_sc as plsc

NEIGHBOUR_IOU = 0.2

_N_DT = 2000
_N_GT = 5000
_ROW_TILE = 200
_NC, _NS, _L = 2, 16, 16
_ROWS_PER_W = 64
_RB = 4
_RING = 4


def _tc_kernel(dt_ref, gtc_ref, dtc_ref, dtgt_ref, mask_ref):
    d = dt_ref[...]
    x1r = d[:, 0:1]
    y1r = d[:, 1:2]
    x2r = d[:, 2:3]
    y2r = d[:, 3:4]
    ar = (x2r - x1r) * (y2r - y1r)

    def tile_parts(c):
        ix1 = jnp.maximum(x1r, c[0:1, :])
        iy1 = jnp.maximum(y1r, c[1:2, :])
        ix2 = jnp.minimum(x2r, c[2:3, :])
        iy2 = jnp.minimum(y2r, c[3:4, :])
        inter = jnp.maximum(ix2 - ix1, 0.0) * jnp.maximum(iy2 - iy1, 0.0)
        union = ar + c[4:5, :] - inter
        return inter, union

    inter_g, union_g = tile_parts(gtc_ref[...])
    dtgt_ref[...] = inter_g / union_g
    inter_d, union_d = tile_parts(dtc_ref[...])
    mask_ref[...] = inter_d >= NEIGHBOUR_IOU * union_d


def _sc_body(x1h, y1h, x2h, y2h, arh, out_hbm,
             x1v, y1v, x2v, y2v, arv, ring, sem):
    wid = lax.axis_index("c") * _NS + lax.axis_index("s")
    base = wid * _ROWS_PER_W
    nrows = jnp.clip(_N_DT - base, 0, _ROWS_PER_W)
    ngroups = nrows // _RB

    pltpu.sync_copy(x1h, x1v)
    pltpu.sync_copy(y1h, y1v)
    pltpu.sync_copy(x2h, x2v)
    pltpu.sync_copy(y2h, y2v)
    pltpu.sync_copy(arh, arv)

    nvec = _N_DT // _L

    def group_body(g, _):
        row0 = base + g * _RB
        slot = lax.rem(g, _RING)

        @pl.when(g >= _RING)
        def _wait():
            pltpu.make_async_copy(
                out_hbm.at[pl.ds(0, _RB)], ring.at[0], sem).wait()

        def splat(ref, row):
            v = ref[pl.ds(row, _L)]
            return jnp.full((_L,), v[0], jnp.float32)

        rx1 = [splat(x1v, row0 + r) for r in range(_RB)]
        ry1 = [splat(y1v, row0 + r) for r in range(_RB)]
        rx2 = [splat(x2v, row0 + r) for r in range(_RB)]
        ry2 = [splat(y2v, row0 + r) for r in range(_RB)]
        rar = [splat(arv, row0 + r) for r in range(_RB)]

        @plsc.parallel_loop(0, nvec, unroll=2)
        def col_body(j):
            s = pl.ds(j * _L, _L)
            cx1 = x1v[s]
            cy1 = y1v[s]
            cx2 = x2v[s]
            cy2 = y2v[s]
            car = arv[s]
            for r in range(_RB):
                ix1 = jnp.maximum(rx1[r], cx1)
                iy1 = jnp.maximum(ry1[r], cy1)
                ix2 = jnp.minimum(rx2[r], cx2)
                iy2 = jnp.minimum(ry2[r], cy2)
                inter = (jnp.maximum(ix2 - ix1, 0.0)
                         * jnp.maximum(iy2 - iy1, 0.0))
                union = rar[r] + car - inter
                ring[slot, r, s] = inter / union

        pltpu.async_copy(ring.at[slot], out_hbm.at[pl.ds(row0, _RB)], sem)
        return 0

    lax.fori_loop(0, ngroups, group_body, 0)

    def drain_body(k, _):
        @pl.when(k < jnp.minimum(ngroups, _RING))
        def _d():
            pltpu.make_async_copy(
                out_hbm.at[pl.ds(0, _RB)], ring.at[0], sem).wait()
        return 0

    lax.fori_loop(0, _RING, drain_body, 0)


def _sc_dtdt(x1, y1, x2, y2, ar):
    mesh = plsc.VectorSubcoreMesh(
        core_axis_name="c", subcore_axis_name="s",
        num_cores=_NC, num_subcores=_NS)
    f = pl.kernel(
        _sc_body,
        out_type=jax.ShapeDtypeStruct((_N_DT, _N_DT), jnp.float32),
        mesh=mesh,
        scratch_types=[
            pltpu.VMEM((2048,), jnp.float32),
            pltpu.VMEM((2048,), jnp.float32),
            pltpu.VMEM((2048,), jnp.float32),
            pltpu.VMEM((2048,), jnp.float32),
            pltpu.VMEM((2048,), jnp.float32),
            pltpu.VMEM((_RING, _RB, _N_DT), jnp.float32),
            pltpu.SemaphoreType.DMA,
        ],
    )
    return f(x1, y1, x2, y2, ar)


def kernel(detections, gt_boxes):
    dt = detections[:_N_DT]

    def cols(b, n):
        x1, y1, x2, y2 = b[:, 0], b[:, 1], b[:, 2], b[:, 3]
        area = (x2 - x1) * (y2 - y1)
        z = jnp.zeros_like(x1)
        return jnp.stack([x1, y1, x2, y2, area, z, z, z], axis=0)

    gtc = cols(gt_boxes, _N_GT)
    dtc = cols(dt, _N_DT)

    pad = 2048 - _N_DT
    x1p = jnp.pad(dtc[0], (0, pad))
    y1p = jnp.pad(dtc[1], (0, pad))
    x2p = jnp.pad(dtc[2], (0, pad))
    y2p = jnp.pad(dtc[3], (0, pad))
    arp = jnp.pad(dtc[4], (0, pad))

    dtdt = _sc_dtdt(x1p, y1p, x2p, y2p, arp)

    br = _ROW_TILE
    dtgt, mask = pl.pallas_call(
        _tc_kernel,
        grid=(_N_DT // br,),
        in_specs=[
            pl.BlockSpec((br, 4), lambda i: (i, 0)),
            pl.BlockSpec((8, _N_GT), lambda i: (0, 0)),
            pl.BlockSpec((8, _N_DT), lambda i: (0, 0)),
        ],
        out_specs=[
            pl.BlockSpec((br, _N_GT), lambda i: (i, 0)),
            pl.BlockSpec((br, _N_DT), lambda i: (i, 0)),
        ],
        out_shape=[
            jax.ShapeDtypeStruct((_N_DT, _N_GT), jnp.float32),
            jax.ShapeDtypeStruct((_N_DT, _N_DT), jnp.bool_),
        ],
    )(dt, gtc, dtc)
    return dtgt, dtdt, mask


# speedup vs baseline: 1.0585x; 1.0585x over previous
import jax
import jax.numpy as jnp
from jax import lax
from jax.experimental import pallas as pl

NEIGHBOUR_IOU = 0.2

_N_DT = 2000
_N_GT = 5000
_BR = 400
_PK = 16


def _tc_kernel(dt_ref, gtc_ref, dtc_ref, dtgt_ref, dtdt_ref, bits_ref):
    d = dt_ref[...]
    x1r = d[:, 0:1]
    y1r = d[:, 1:2]
    x2r = d[:, 2:3]
    y2r = d[:, 3:4]
    ar = (x2r - x1r) * (y2r - y1r)

    def tile_parts(c):
        ix1 = jnp.maximum(x1r, c[0:1, :])
        iy1 = jnp.maximum(y1r, c[1:2, :])
        ix2 = jnp.minimum(x2r, c[2:3, :])
        iy2 = jnp.minimum(y2r, c[3:4, :])
        inter = jnp.maximum(ix2 - ix1, 0.0) * jnp.maximum(iy2 - iy1, 0.0)
        union = ar + c[4:5, :] - inter
        return inter, union

    inter_g, union_g = tile_parts(gtc_ref[...])
    dtgt_ref[...] = inter_g * pl.reciprocal(union_g, approx=True)

    inter_d, union_d = tile_parts(dtc_ref[...])
    dtdt_ref[...] = inter_d * pl.reciprocal(union_d, approx=True)

    cmpf = (inter_d >= NEIGHBOUR_IOU * union_d).astype(jnp.float32)
    g = lax.broadcasted_iota(jnp.int32, (_BR // _PK, _BR), 0)
    r = lax.broadcasted_iota(jnp.int32, (_BR // _PK, _BR), 1)
    w = jnp.where(r // _PK == g, 1 << (r % _PK), 0).astype(jnp.float32)
    bits = jnp.dot(w, cmpf, preferred_element_type=jnp.float32)
    bits_ref[0] = bits.astype(jnp.int32)


def kernel(detections, gt_boxes):
    dt = detections[:_N_DT]

    def cols(b):
        x1, y1, x2, y2 = b[:, 0], b[:, 1], b[:, 2], b[:, 3]
        area = (x2 - x1) * (y2 - y1)
        z = jnp.zeros_like(x1)
        return jnp.stack([x1, y1, x2, y2, area, z, z, z], axis=0)

    gtc = cols(gt_boxes)
    dtc = cols(dt)

    dtgt, dtdt, bits = pl.pallas_call(
        _tc_kernel,
        grid=(_N_DT // _BR,),
        in_specs=[
            pl.BlockSpec((_BR, 4), lambda i: (i, 0)),
            pl.BlockSpec((8, _N_GT), lambda i: (0, 0)),
            pl.BlockSpec((8, _N_DT), lambda i: (0, 0)),
        ],
        out_specs=[
            pl.BlockSpec((_BR, _N_GT), lambda i: (i, 0)),
            pl.BlockSpec((_BR, _N_DT), lambda i: (i, 0)),
            pl.BlockSpec((1, _BR // _PK, _N_DT), lambda i: (i, 0, 0)),
        ],
        out_shape=[
            jax.ShapeDtypeStruct((_N_DT, _N_GT), jnp.float32),
            jax.ShapeDtypeStruct((_N_DT, _N_DT), jnp.float32),
            jax.ShapeDtypeStruct(
                (_N_DT // _BR, _BR // _PK, _N_DT), jnp.int32),
        ],
    )(dt, gtc, dtc)

    sh = jnp.arange(_PK, dtype=jnp.int32).reshape(1, 1, _PK, 1)
    mask = ((bits[:, :, None, :] >> sh) & 1).astype(jnp.bool_)
    mask = mask.reshape(_N_DT, _N_DT)
    return dtgt, dtdt, mask
